# baseline (speedup 1.0000x reference)
import functools

import jax
import jax.numpy as jnp
from jax import lax
from jax.experimental import pallas as pl
from jax.experimental.pallas import tpu as pltpu
from jax.experimental.pallas import tpu_sc as plsc

_B = 16384
_C = 128
_NC = 2
_NS = 16
_NW = _NC * _NS
_T = 128
_NT = _B // (_T * _NW)
_G = _C // 16


def _body(params_hbm, idx_hbm, out_hbm, idx_v, ridx_v,
          in_v0, in_v1, st_v0, st_v1, sem_in, sem_out):
    wid = lax.axis_index("s") * _NC + lax.axis_index("c")

    lane = jnp.arange(16, dtype=jnp.int32)
    rows_g = [g * 16 + lane for g in range(_G)]

    in_bufs = [in_v0, in_v1]
    st_bufs = [st_v0, st_v1]

    def start_in(t):
        row0 = (wid * _NT + t) * _T
        return pltpu.async_copy(
            params_hbm.at[pl.ds(row0, _T)], in_bufs[t % 2], sem_in)

    in_descs = [start_in(0)]
    idx_desc = pltpu.async_copy(idx_hbm, idx_v, sem_out)

    for t in range(_NT):
        tb2 = (wid * _NT + t) * 2
        for g in range(_G):
            cvec = rows_g[g]
            r = (cvec >> 1) * 256 + tb2 + (cvec & 1)
            plsc.store_scatter(ridx_v, [jnp.full((16,), t, jnp.int32), cvec], r)

    idx_desc.wait()

    idx_g = [
        plsc.load_gather(
            idx_v, [(g * 16 + lane) >> 1, (g * 16 + lane) & 1])
        for g in range(_G)
    ]
    cols_g = rows_g

    def compute(t):
        in_v = in_bufs[t % 2]
        st_v = st_bufs[t % 2]

        @plsc.parallel_loop(0, _G * 16, 1, unroll=4)
        def diag_body(m):
            rowv = (m >> 4) * 16 + ((lane + (m & 15)) & 15)
            for cb in range(_G):
                vals = plsc.load_gather(in_v, [rowv, idx_g[cb]])
                plsc.store_scatter(st_v, [cols_g[cb], rowv], vals)

    def start_out(t):
        return pltpu.async_copy(
            st_bufs[t % 2], out_hbm.at[ridx_v.at[t]], sem_out)

    out_descs = []
    for t in range(_NT):
        in_descs[t].wait()
        if t + 1 < _NT:
            in_descs.append(start_in(t + 1))
        compute(t)
        if t >= 2:
            out_descs[t - 2].wait()
        out_descs.append(start_out(t))
    out_descs[_NT - 2].wait()
    out_descs[_NT - 1].wait()


_sc_call = functools.partial(
    pl.kernel,
    out_type=jax.ShapeDtypeStruct((_B, _C), jnp.float32),
    mesh=plsc.VectorSubcoreMesh(core_axis_name="c", subcore_axis_name="s"),
    scratch_types=[
        pltpu.VMEM((_C // 2, 2), jnp.int32),
        pltpu.VMEM((_NT, _C), jnp.int32),
        pltpu.VMEM((_T, _C), jnp.float32),
        pltpu.VMEM((_T, _C), jnp.float32),
        pltpu.VMEM((_T, _C), jnp.float32),
        pltpu.VMEM((_T, _C), jnp.float32),
        pltpu.SemaphoreType.DMA,
        pltpu.SemaphoreType.DMA,
    ],
    compiler_params=pltpu.CompilerParams(
        needs_layout_passes=False, use_tc_tiling_on_sc=False,
        disable_bounds_checks=True, disable_semaphore_checks=True),
)(_body)


@jax.jit
def kernel(parameters, marginal_indices):
    r = _sc_call(parameters, marginal_indices)
    r4 = r.reshape(_C // 2, _B // _T, 2, _T)
    return r4.transpose(1, 3, 0, 2).reshape(_B, _C // 2, 2)


# speedup vs baseline: 1.0277x; 1.0277x over previous
import functools

import jax
import jax.numpy as jnp
from jax import lax
from jax.experimental import pallas as pl
from jax.experimental.pallas import tpu as pltpu
from jax.experimental.pallas import tpu_sc as plsc

_B = 16384
_C = 128
_NC = 2
_NS = 16
_NW = _NC * _NS
_T = 128
_NT = _B // (_T * _NW)
_G = _C // 16


def _body(params_hbm, idx_hbm, out_hbm, idx_v, ridx_v,
          in_v, st_v, sem_in, sem_out):
    wid = lax.axis_index("s") * _NC + lax.axis_index("c")

    lane = jnp.arange(16, dtype=jnp.int32)
    rows_g = [g * 16 + lane for g in range(_G)]

    def start_in(t):
        row0 = (wid * _NT + t) * _T
        return pltpu.async_copy(
            params_hbm.at[pl.ds(row0, _T)], in_v.at[t & 1], sem_in)

    start_in(0)
    idx_desc = pltpu.async_copy(idx_hbm, idx_v, sem_out)

    for t in range(_NT):
        tb2 = (wid * _NT + t) * 2
        for g in range(_G):
            cvec = rows_g[g]
            r = (cvec >> 1) * 256 + tb2 + (cvec & 1)
            plsc.store_scatter(ridx_v, [jnp.full((16,), t, jnp.int32), cvec], r)

    idx_desc.wait()

    idx_g = [
        plsc.load_gather(
            idx_v, [(g * 16 + lane) >> 1, (g * 16 + lane) & 1])
        for g in range(_G)
    ]
    cols_g = rows_g

    def wait_in(t):
        pltpu.make_async_copy(
            params_hbm.at[pl.ds(0, _T)], in_v.at[t & 1], sem_in).wait()

    def start_out(t):
        return pltpu.async_copy(
            st_v.at[t & 1], out_hbm.at[ridx_v.at[t]], sem_out)

    def wait_out(t):
        pltpu.make_async_copy(
            st_v.at[t & 1], out_hbm.at[ridx_v.at[0]], sem_out).wait()

    def tile_body(t, carry):
        p = t & 1
        wait_in(t)

        @pl.when(t + 1 < _NT)
        def _():
            start_in(t + 1)

        pv = jnp.full((16,), p, jnp.int32)

        @plsc.parallel_loop(0, _G * 16, 1, unroll=2)
        def diag_body(m):
            rowv = (m >> 4) * 16 + ((lane + (m & 15)) & 15)
            for cb in range(_G):
                vals = plsc.load_gather(in_v, [pv, rowv, idx_g[cb]])
                plsc.store_scatter(st_v, [pv, cols_g[cb], rowv], vals)

        @pl.when(t >= 2)
        def _():
            wait_out(t)

        start_out(t)
        return carry

    lax.fori_loop(0, _NT, tile_body, 0)
    wait_out(_NT - 2)
    wait_out(_NT - 1)


_sc_call = functools.partial(
    pl.kernel,
    out_type=jax.ShapeDtypeStruct((_B, _C), jnp.float32),
    mesh=plsc.VectorSubcoreMesh(core_axis_name="c", subcore_axis_name="s"),
    scratch_types=[
        pltpu.VMEM((_C // 2, 2), jnp.int32),
        pltpu.VMEM((_NT, _C), jnp.int32),
        pltpu.VMEM((2, _T, _C), jnp.float32),
        pltpu.VMEM((2, _T, _C), jnp.float32),
        pltpu.SemaphoreType.DMA,
        pltpu.SemaphoreType.DMA,
    ],
    compiler_params=pltpu.CompilerParams(
        needs_layout_passes=False, use_tc_tiling_on_sc=False,
        disable_bounds_checks=True, disable_semaphore_checks=True),
)(_body)


@jax.jit
def kernel(parameters, marginal_indices):
    r = _sc_call(parameters, marginal_indices)
    r4 = r.reshape(_C // 2, _B // _T, 2, _T)
    return r4.transpose(1, 3, 0, 2).reshape(_B, _C // 2, 2)


# speedup vs baseline: 1.0434x; 1.0153x over previous
import functools

import jax
import jax.numpy as jnp
from jax import lax
from jax.experimental import pallas as pl
from jax.experimental.pallas import tpu as pltpu
from jax.experimental.pallas import tpu_sc as plsc

_B = 16384
_C = 128
_NC = 2
_NS = 16
_NW = _NC * _NS
_T = 128
_NT = _B // (_T * _NW)
_G = _C // 16


def _body(params_hbm, idx_hbm, out_hbm, idx_v, ridx_v,
          in_v, st_v, sem_in, sem_out):
    wid = lax.axis_index("s") * _NC + lax.axis_index("c")

    lane = jnp.arange(16, dtype=jnp.int32)
    rows_g = [g * 16 + lane for g in range(_G)]

    def start_in(t):
        row0 = (wid * _NT + t) * _T
        return pltpu.async_copy(
            params_hbm.at[pl.ds(row0, _T)], in_v.at[t & 1], sem_in)

    start_in(0)
    idx_desc = pltpu.async_copy(idx_hbm, idx_v, sem_out)

    for t in range(_NT):
        tb = wid * _NT + t
        for g in range(_G // 2):
            ivec = g * 16 + lane
            plsc.store_scatter(
                ridx_v, [jnp.full((16,), t, jnp.int32), ivec], ivec * 128 + tb)

    idx_desc.wait()

    idx_g = [
        plsc.load_gather(
            idx_v, [(g * 16 + lane) >> 1, (g * 16 + lane) & 1])
        for g in range(_G)
    ]
    cols_g = rows_g
    st_i_g = [c >> 1 for c in cols_g]
    st_o_g = [(c & 1) * 128 for c in cols_g]

    def wait_in(t):
        pltpu.make_async_copy(
            params_hbm.at[pl.ds(0, _T)], in_v.at[t & 1], sem_in).wait()

    def start_out(t):
        return pltpu.async_copy(
            st_v.at[t & 1], out_hbm.at[ridx_v.at[t]], sem_out)

    def wait_out(t):
        pltpu.make_async_copy(
            st_v.at[t & 1], out_hbm.at[ridx_v.at[0]], sem_out).wait()

    def tile_body(t, carry):
        p = t & 1
        wait_in(t)

        @pl.when(t + 1 < _NT)
        def _():
            start_in(t + 1)

        pv = jnp.full((16,), p, jnp.int32)

        @plsc.parallel_loop(0, _G * 16, 1, unroll=2)
        def diag_body(m):
            rowv = (m >> 4) * 16 + ((lane + (m & 15)) & 15)
            for cb in range(_G):
                vals = plsc.load_gather(in_v, [pv, rowv, idx_g[cb]])
                plsc.store_scatter(
                    st_v, [pv, st_i_g[cb], st_o_g[cb] + rowv], vals)

        @pl.when(t >= 2)
        def _():
            wait_out(t)

        start_out(t)
        return carry

    lax.fori_loop(0, _NT, tile_body, 0)
    wait_out(_NT - 2)
    wait_out(_NT - 1)


_sc_call = functools.partial(
    pl.kernel,
    out_type=jax.ShapeDtypeStruct((_B // 2, _C * 2), jnp.float32),
    mesh=plsc.VectorSubcoreMesh(core_axis_name="c", subcore_axis_name="s"),
    scratch_types=[
        pltpu.VMEM((_C // 2, 2), jnp.int32),
        pltpu.VMEM((_NT, _C // 2), jnp.int32),
        pltpu.VMEM((2, _T, _C), jnp.float32),
        pltpu.VMEM((2, _C // 2, _T * 2), jnp.float32),
        pltpu.SemaphoreType.DMA,
        pltpu.SemaphoreType.DMA,
    ],
    compiler_params=pltpu.CompilerParams(
        needs_layout_passes=False, use_tc_tiling_on_sc=False,
        disable_bounds_checks=True, disable_semaphore_checks=True),
)(_body)


@jax.jit
def kernel(parameters, marginal_indices):
    r = _sc_call(parameters, marginal_indices)
    r4 = r.reshape(_C // 2, _B // _T, 2, _T)
    return r4.transpose(1, 3, 0, 2).reshape(_B, _C // 2, 2)


# speedup vs baseline: 1.0499x; 1.0062x over previous
import functools

import jax
import jax.numpy as jnp
from jax import lax
from jax.experimental import pallas as pl
from jax.experimental.pallas import tpu as pltpu
from jax.experimental.pallas import tpu_sc as plsc

_B = 16384
_C = 128
_NC = 2
_NS = 16
_NW = _NC * _NS
_T = 128
_NT = _B // (_T * _NW)
_G = _C // 16


def _body(params_hbm, idx_hbm, out_hbm, idx_v, ridx_v,
          in_v, st_v, sem_in, sem_out):
    wid = lax.axis_index("s") * _NC + lax.axis_index("c")

    lane = jnp.arange(16, dtype=jnp.int32)
    rows_g = [g * 16 + lane for g in range(_G)]

    def start_in(t):
        row0 = (wid * _NT + t) * _T
        return pltpu.async_copy(
            params_hbm.at[pl.ds(row0, _T)], in_v.at[t & 1], sem_in)

    start_in(0)
    idx_desc = pltpu.async_copy(idx_hbm, idx_v, sem_out)

    for t in range(_NT):
        tb = wid * _NT + t
        for g in range(_G // 2):
            ivec = g * 16 + lane
            plsc.store_scatter(
                ridx_v, [jnp.full((16,), t, jnp.int32), ivec], ivec * 128 + tb)

    idx_desc.wait()

    idx_g = [
        plsc.load_gather(idx_v, [g * 16 + lane]) for g in range(_G)
    ]
    cols_g = rows_g
    st_i_g = [c >> 1 for c in cols_g]
    st_o_g = [(c & 1) * 128 for c in cols_g]

    def wait_in(t):
        pltpu.make_async_copy(
            params_hbm.at[pl.ds(0, _T)], in_v.at[t & 1], sem_in).wait()

    def start_out(t):
        return pltpu.async_copy(
            st_v.at[t & 1], out_hbm.at[ridx_v.at[t]], sem_out)

    def wait_out(t):
        pltpu.make_async_copy(
            st_v.at[t & 1], out_hbm.at[ridx_v.at[0]], sem_out).wait()

    def tile_body(t, carry):
        p = t & 1
        wait_in(t)

        @pl.when(t + 1 < _NT)
        def _():
            start_in(t + 1)

        pv = jnp.full((16,), p, jnp.int32)

        @plsc.parallel_loop(0, _G * 16, 1, unroll=2)
        def diag_body(m):
            rowv = (m >> 4) * 16 + ((lane + (m & 15)) & 15)
            for cb in range(_G):
                vals = plsc.load_gather(in_v, [pv, rowv, idx_g[cb]])
                plsc.store_scatter(
                    st_v, [pv, st_i_g[cb], st_o_g[cb] + rowv], vals)

        @pl.when(t >= 2)
        def _():
            wait_out(t)

        start_out(t)
        return carry

    lax.fori_loop(0, _NT, tile_body, 0)
    wait_out(_NT - 2)
    wait_out(_NT - 1)


_sc_call = functools.partial(
    pl.kernel,
    out_type=jax.ShapeDtypeStruct((_B // 2, _C * 2), jnp.float32),
    mesh=plsc.VectorSubcoreMesh(core_axis_name="c", subcore_axis_name="s"),
    scratch_types=[
        pltpu.VMEM((_C,), jnp.int32),
        pltpu.VMEM((_NT, _C // 2), jnp.int32),
        pltpu.VMEM((2, _T, _C), jnp.float32),
        pltpu.VMEM((2, _C // 2, _T * 2), jnp.float32),
        pltpu.SemaphoreType.DMA,
        pltpu.SemaphoreType.DMA,
    ],
    compiler_params=pltpu.CompilerParams(
        needs_layout_passes=False, use_tc_tiling_on_sc=False,
        disable_bounds_checks=True, disable_semaphore_checks=True),
)(_body)


@jax.jit
def kernel(parameters, marginal_indices):
    r = _sc_call(parameters, marginal_indices.reshape(_C))
    r4 = r.reshape(_C // 2, _B // _T, 2, _T)
    return r4.transpose(1, 3, 0, 2).reshape(_B, _C // 2, 2)
